# Initial kernel scaffold; baseline (speedup 1.0000x reference)
import jax
import jax.numpy as jnp
from jax.experimental import pallas as pl


def kernel(x, edge_index, W1, b1, tc1_w, tc1_b, bn1_g, bn1_b, bn1_rm, bn1_rv, W2, b2, tc2_w, tc2_b, bn2_g, bn2_b, bn2_rm, bn2_rv, fc1_w, fc1_b, fc2_w, fc2_b):
    raise NotImplementedError("write your pallas kernel here")


# speedup vs baseline: 25.1349x; 25.1349x over previous
import functools

import jax
import jax.numpy as jnp
from jax import lax
from jax.experimental import pallas as pl
from jax.experimental.pallas import tpu as pltpu
from jax.experimental.pallas import tpu_sc as plsc

N_REAL = 75
NP = 80
T = 640
NB = 8
TB = 80
HALO = 4
W = TB + 2 * HALO
HID = 128
NE_PAD = 160
PAD_NODE = 79


NE_TOT = NE_PAD + NP
NC = 128


def _build_counts_sc(src_all, dst_all, eye_np, zeros_np):
    mesh = plsc.VectorSubcoreMesh(core_axis_name="c", subcore_axis_name="s")

    @functools.partial(
        pl.kernel,
        mesh=mesh,
        out_type=jax.ShapeDtypeStruct((NP, NC), jnp.float32),
        scratch_types=[
            pltpu.VMEM((NE_TOT,), jnp.int32),
            pltpu.VMEM((NE_TOT,), jnp.int32),
            pltpu.VMEM((NE_TOT, NC), jnp.float32),
            pltpu.MemorySpace.VMEM_SHARED((NP, NC), jnp.float32),
        ],
    )
    def k(src_hbm, dst_hbm, eye_hbm, zero_hbm, out_hbm,
          src_v, dst_v, oh_v, c_sh):
        is_w0 = jnp.logical_and(
            lax.axis_index("c") == 0, lax.axis_index("s") == 0
        )

        @pl.when(is_w0)
        def _():
            pltpu.sync_copy(src_hbm, src_v)
            pltpu.sync_copy(dst_hbm, dst_v)
            pltpu.sync_copy(zero_hbm, c_sh)
            pltpu.sync_copy(eye_hbm.at[src_v], oh_v)
            pltpu.sync_copy(oh_v, c_sh.at[dst_v], add=True)
            pltpu.sync_copy(c_sh, out_hbm)

    return k(src_all, dst_all, eye_np, zeros_np)


def _adj_body(c_ref, a_ref):
    c = c_ref[:, :NP]
    deg = jnp.sum(c, axis=1, keepdims=True)
    dinv = jnp.where(deg > 0, lax.rsqrt(deg), 0.0)
    a_ref[...] = c * dinv * jnp.transpose(dinv)


def _normalize_adj(counts):
    return pl.pallas_call(
        _adj_body,
        out_shape=jax.ShapeDtypeStruct((NP, NP), jnp.float32),
    )(counts)


def _main_body(x_ref, a_ref, w1_ref, b1_ref, t1_ref, t1b_ref,
               g1_ref, be1_ref, rm1_ref, rv1_ref,
               w2_ref, b2_ref, t2_ref, t2b_ref,
               g2_ref, be2_ref, rm2_ref, rv2_ref,
               f1_ref, f1b_ref, f2_ref, f2b_ref, out_ref):
    b = pl.program_id(0)
    A = a_ref[...]

    row = lax.broadcasted_iota(jnp.int32, (NP * W, 1), 0)
    wv = row % W
    gt = wv + (b * TB - HALO)
    m_t = jnp.logical_and(gt >= 0, gt < T).astype(jnp.float32)

    sc1 = g1_ref[...] * lax.rsqrt(rv1_ref[...] + 1e-5)
    sh1 = be1_ref[...] - rm1_ref[...] * sc1
    sc2 = g2_ref[...] * lax.rsqrt(rv2_ref[...] + 1e-5)
    sh2 = be2_ref[...] - rm2_ref[...] * sc2

    def layer(h, w_ref, b_ref, t_ref, tb_ref, sc, sh):
        xw = jnp.dot(h, w_ref[...], preferred_element_type=jnp.float32)
        xwm = jnp.reshape(xw, (NP, W * HID))
        gm = jnp.dot(A, xwm, preferred_element_type=jnp.float32)
        g = jnp.reshape(gm, (NP * W, HID)) + b_ref[...]
        g = g * m_t
        y0 = jnp.dot(g, t_ref[0], preferred_element_type=jnp.float32)
        y1 = jnp.dot(g, t_ref[1], preferred_element_type=jnp.float32)
        y2 = jnp.dot(g, t_ref[2], preferred_element_type=jnp.float32)
        c = (pltpu.roll(y0, 1, 0) + y1 + pltpu.roll(y2, NP * W - 1, 0)
             + tb_ref[...])
        return jnp.maximum(c * sc + sh, 0.0)

    x = jnp.reshape(x_ref[0], (NP * W, 8))
    h1 = layer(x, w1_ref, b1_ref, t1_ref, t1b_ref, sc1, sh1)
    h2 = layer(h1, w2_ref, b2_ref, t2_ref, t2b_ref, sc2, sh2)

    nv = row // W
    m_head = m_t * jnp.logical_and(
        jnp.logical_and(wv >= HALO, wv < HALO + TB), nv < N_REAL
    ).astype(jnp.float32)
    s = jnp.sum(h2 * m_head, axis=0, keepdims=True) * (1.0 / (N_REAL * TB))
    z = jnp.maximum(
        jnp.dot(s, f1_ref[...], preferred_element_type=jnp.float32)
        + f1b_ref[...], 0.0)
    o = (jnp.dot(z, f2_ref[...], preferred_element_type=jnp.float32)
         + f2b_ref[...])
    out_ref[...] = jnp.reshape(o, (1, 1, 100))


def _run_main(xwin, A, w1p, b1, tc1T, tc1b, bn1, w2, b2, tc2T, tc2b, bn2,
              fc1_w, fc1_b, fc2_w, fc2_b, interpret=False):
    g1, be1, rm1, rv1 = bn1
    g2, be2, rm2, rv2 = bn2
    full = lambda shape: pl.BlockSpec(shape, lambda b: (0,) * len(shape))
    return pl.pallas_call(
        _main_body,
        grid=(NB,),
        in_specs=[
            pl.BlockSpec((1, NP, W, 8), lambda b: (b, 0, 0, 0)),
            full((NP, NP)),
            full((8, HID)), full((1, HID)), full((3, HID, HID)),
            full((1, HID)),
            full((1, HID)), full((1, HID)), full((1, HID)), full((1, HID)),
            full((HID, HID)), full((1, HID)), full((3, HID, HID)),
            full((1, HID)),
            full((1, HID)), full((1, HID)), full((1, HID)), full((1, HID)),
            full((HID, 64)), full((1, 64)), full((64, 100)), full((1, 100)),
        ],
        out_specs=pl.BlockSpec((1, 1, 100), lambda b: (b, 0, 0)),
        out_shape=jax.ShapeDtypeStruct((NB, 1, 100), jnp.float32),
        interpret=interpret,
    )(xwin, A, w1p, b1, tc1T, tc1b, g1, be1, rm1, rv1,
      w2, b2, tc2T, tc2b, g2, be2, rm2, rv2,
      fc1_w, fc1_b, fc2_w, fc2_b).reshape(NB, 100)


def _prep_x(x):
    xf = x.reshape(T, N_REAL, 3)
    xp = jnp.pad(xf, ((HALO, HALO), (0, NP - N_REAL), (0, 5)))
    xwin = jnp.stack([xp[b * TB:b * TB + W] for b in range(NB)])
    return jnp.transpose(xwin, (0, 2, 1, 3))


def kernel(x, edge_index, W1, b1, tc1_w, tc1_b, bn1_g, bn1_b, bn1_rm, bn1_rv,
           W2, b2, tc2_w, tc2_b, bn2_g, bn2_b, bn2_rm, bn2_rv,
           fc1_w, fc1_b, fc2_w, fc2_b):
    pad_e = jnp.full((NE_PAD - edge_index.shape[1],), PAD_NODE, jnp.int32)
    loops = jnp.arange(NP, dtype=jnp.int32)
    src = jnp.concatenate([edge_index[0], pad_e, loops])
    dst = jnp.concatenate([edge_index[1], pad_e, loops])
    counts = _build_counts_sc(
        src, dst, jnp.eye(NP, NC, dtype=jnp.float32),
        jnp.zeros((NP, NC), jnp.float32))
    A = _normalize_adj(counts)

    xwin = _prep_x(x)
    w1p = jnp.pad(W1, ((0, 5), (0, 0)))
    tc1T = jnp.transpose(tc1_w, (2, 1, 0))
    tc2T = jnp.transpose(tc2_w, (2, 1, 0))
    r = lambda v: v.reshape(1, -1)
    return _run_main(
        xwin, A, w1p, r(b1), tc1T, r(tc1_b),
        (r(bn1_g), r(bn1_b), r(bn1_rm), r(bn1_rv)),
        W2, r(b2), tc2T, r(tc2_b),
        (r(bn2_g), r(bn2_b), r(bn2_rm), r(bn2_rv)),
        fc1_w, r(fc1_b), fc2_w, r(fc2_b))
